# baseline (speedup 1.0000x reference)
import functools
import math

import jax
import jax.numpy as jnp
from jax import lax
from jax.experimental import pallas as pl
from jax.experimental.pallas import tpu as pltpu
from jax.experimental.pallas import tpu_sc as plsc

_NTOKEN = 100000
_NINP = 1024
_SEQ = 4096
_BATCH = 4

_PART = (256, 1792, 1792, 256)

_NC = 2
_NS = 16
_NW = _NC * _NS
_CHUNK = 32


def _sc_gather_body(rows_per_w, table_hbm, idx_hbm, out_hbm,
                    idx_v, rows0, rows1, gsem0, gsem1):
    wid = lax.axis_index("s") * _NC + lax.axis_index("c")
    base = wid * rows_per_w
    nchunk = rows_per_w // _CHUNK
    bufs = (rows0, rows1)
    sems = (gsem0, gsem1)
    pltpu.sync_copy(idx_hbm.at[pl.ds(base, rows_per_w)], idx_v)

    def _gather(c):
        return pltpu.async_copy(
            table_hbm.at[idx_v.at[pl.ds(c * _CHUNK, _CHUNK)]],
            bufs[c % 2], sems[c % 2])

    copies = {0: _gather(0)}
    for c in range(nchunk):
        if c + 1 < nchunk:
            copies[c + 1] = _gather(c + 1)
        copies[c].wait()
        pltpu.sync_copy(bufs[c % 2],
                        out_hbm.at[pl.ds(base + c * _CHUNK, _CHUNK)])


@functools.cache
def _sc_gather(nrows):
    rows_per_w = nrows // _NW
    return pl.kernel(
        functools.partial(_sc_gather_body, rows_per_w),
        mesh=plsc.VectorSubcoreMesh(core_axis_name="c", subcore_axis_name="s"),
        out_type=jax.ShapeDtypeStruct((nrows, _NINP), jnp.float32),
        scratch_types=[
            pltpu.VMEM((rows_per_w,), jnp.int32),
            pltpu.VMEM((_CHUNK, _NINP), jnp.float32),
            pltpu.VMEM((_CHUNK, _NINP), jnp.float32),
            pltpu.SemaphoreType.DMA,
            pltpu.SemaphoreType.DMA,
        ],
    )


_S_BLK = 256


def _logsoftmax_body(g_ref, pe_ref, o_ref):
    pe3 = pe_ref[...]
    pe_exp = jnp.broadcast_to(
        pe3, (_S_BLK, _BATCH, _NINP)
    ).reshape(_S_BLK * _BATCH, _NINP)
    y = g_ref[...] * math.sqrt(_NINP) + pe_exp
    m = jnp.max(y, axis=-1, keepdims=True)
    e = jnp.exp(y - m)
    s = jnp.sum(e, axis=-1, keepdims=True)
    out2 = y - m - jnp.log(s)
    o_ref[...] = out2.reshape(_S_BLK, _BATCH, _NINP)


def _tc_logsoftmax_chunk(g2, pe, prev, seq_off, seq_len):
    blk_off = seq_off // _S_BLK
    operands = [g2, pe] if prev is None else [g2, pe, prev]
    in_specs = [
        pl.BlockSpec((_S_BLK * _BATCH, _NINP), lambda i: (i, 0)),
        pl.BlockSpec((_S_BLK, 1, _NINP),
                     lambda i, _o=blk_off: (_o + i, 0, 0)),
    ]
    if prev is not None:
        in_specs.append(pl.BlockSpec(memory_space=pl.ANY))

    def _body(g_ref, pe_ref, *rest):
        _logsoftmax_body(g_ref, pe_ref, rest[-1])

    return pl.pallas_call(
        _body,
        grid=(seq_len // _S_BLK,),
        in_specs=in_specs,
        out_specs=pl.BlockSpec((_S_BLK, _BATCH, _NINP),
                               lambda i, _o=blk_off: (_o + i, 0, 0)),
        out_shape=jax.ShapeDtypeStruct((_SEQ, _BATCH, _NINP), jnp.float32),
        input_output_aliases={} if prev is None else {2: 0},
    )(*operands)


def kernel(src, emb_weight, pe):
    idx = src.reshape(-1).astype(jnp.int32)
    gathered = []
    row_off = 0
    for seq_len in _PART:
        nrows = seq_len * _BATCH
        gathered.append(_sc_gather(nrows)(
            emb_weight, lax.slice(idx, (row_off,), (row_off + nrows,))))
        row_off += nrows
    out = None
    seq_off = 0
    for g2, seq_len in zip(gathered, _PART):
        out = _tc_logsoftmax_chunk(g2, pe, out, seq_off, seq_len)
        seq_off += seq_len
    return out


# speedup vs baseline: 1.0545x; 1.0545x over previous
import functools
import math

import jax
import jax.numpy as jnp
from jax import lax
from jax.experimental import pallas as pl
from jax.experimental.pallas import tpu as pltpu
from jax.experimental.pallas import tpu_sc as plsc

_NTOKEN = 100000
_NINP = 1024
_SEQ = 4096
_BATCH = 4

_PART = (2048, 2048)

_NC = 2
_NS = 16
_NW = _NC * _NS
_CHUNK = 32


def _sc_gather_body(rows_per_w, table_hbm, idx_hbm, out_hbm,
                    idx_v, rows0, rows1, gsem0, gsem1):
    wid = lax.axis_index("s") * _NC + lax.axis_index("c")
    base = wid * rows_per_w
    nchunk = rows_per_w // _CHUNK
    bufs = (rows0, rows1)
    sems = (gsem0, gsem1)
    pltpu.sync_copy(idx_hbm.at[pl.ds(base, rows_per_w)], idx_v)

    def _gather(c):
        return pltpu.async_copy(
            table_hbm.at[idx_v.at[pl.ds(c * _CHUNK, _CHUNK)]],
            bufs[c % 2], sems[c % 2])

    copies = {0: _gather(0)}
    for c in range(nchunk):
        if c + 1 < nchunk:
            copies[c + 1] = _gather(c + 1)
        copies[c].wait()
        pltpu.sync_copy(bufs[c % 2],
                        out_hbm.at[pl.ds(base + c * _CHUNK, _CHUNK)])


@functools.cache
def _sc_gather(nrows):
    rows_per_w = nrows // _NW
    return pl.kernel(
        functools.partial(_sc_gather_body, rows_per_w),
        mesh=plsc.VectorSubcoreMesh(core_axis_name="c", subcore_axis_name="s"),
        out_type=jax.ShapeDtypeStruct((nrows, _NINP), jnp.float32),
        scratch_types=[
            pltpu.VMEM((rows_per_w,), jnp.int32),
            pltpu.VMEM((_CHUNK, _NINP), jnp.float32),
            pltpu.VMEM((_CHUNK, _NINP), jnp.float32),
            pltpu.SemaphoreType.DMA,
            pltpu.SemaphoreType.DMA,
        ],
    )


_S_BLK = 256


def _logsoftmax_body(g_ref, pe_ref, o_ref):
    pe3 = pe_ref[...]
    pe_exp = jnp.broadcast_to(
        pe3, (_S_BLK, _BATCH, _NINP)
    ).reshape(_S_BLK * _BATCH, _NINP)
    y = g_ref[...] * math.sqrt(_NINP) + pe_exp
    m = jnp.max(y, axis=-1, keepdims=True)
    e = jnp.exp(y - m)
    s = jnp.sum(e, axis=-1, keepdims=True)
    out2 = y - m - jnp.log(s)
    o_ref[...] = out2.reshape(_S_BLK, _BATCH, _NINP)


def _tc_logsoftmax_chunk(g2, pe, prev, seq_off, seq_len):
    blk_off = seq_off // _S_BLK
    operands = [g2, pe] if prev is None else [g2, pe, prev]
    in_specs = [
        pl.BlockSpec((_S_BLK * _BATCH, _NINP), lambda i: (i, 0)),
        pl.BlockSpec((_S_BLK, 1, _NINP),
                     lambda i, _o=blk_off: (_o + i, 0, 0)),
    ]
    if prev is not None:
        in_specs.append(pl.BlockSpec(memory_space=pl.ANY))

    def _body(g_ref, pe_ref, *rest):
        _logsoftmax_body(g_ref, pe_ref, rest[-1])

    return pl.pallas_call(
        _body,
        grid=(seq_len // _S_BLK,),
        in_specs=in_specs,
        out_specs=pl.BlockSpec((_S_BLK, _BATCH, _NINP),
                               lambda i, _o=blk_off: (_o + i, 0, 0)),
        out_shape=jax.ShapeDtypeStruct((_SEQ, _BATCH, _NINP), jnp.float32),
        input_output_aliases={} if prev is None else {2: 0},
    )(*operands)


def kernel(src, emb_weight, pe):
    idx = src.reshape(-1).astype(jnp.int32)
    gathered = []
    row_off = 0
    for seq_len in _PART:
        nrows = seq_len * _BATCH
        gathered.append(_sc_gather(nrows)(
            emb_weight, lax.slice(idx, (row_off,), (row_off + nrows,))))
        row_off += nrows
    out = None
    seq_off = 0
    for g2, seq_len in zip(gathered, _PART):
        out = _tc_logsoftmax_chunk(g2, pe, out, seq_off, seq_len)
        seq_off += seq_len
    return out
